# baseline (speedup 1.0000x reference)
import functools

import jax
import jax.numpy as jnp
from jax import lax
from jax.experimental import pallas as pl
from jax.experimental.pallas import tpu as pltpu
from jax.experimental.pallas import tpu_sc as plsc

MAX_H = 128
P = 2048
D = 512
B = 8
NC = 2
NS = 16
NW = NC * NS
PC = 64
NZCHUNKS = P // PC
NZ = (MAX_H - 1) * NZCHUNKS
PER_W = NZ // NW
MEANP = P // NW
LANES = 16
VECS_PER_ROW = D // LANES


def _sc_kernel(state_hbm, out_hbm, zeros_v, acc_v, chunk_v, zsem, ssem):
    wid = lax.axis_index("s") * NC + lax.axis_index("c")

    def zfill(t, carry):
        r = t // VECS_PER_ROW
        i = t % VECS_PER_ROW
        zeros_v[r, pl.ds(i * LANES, LANES)] = jnp.zeros((LANES,), jnp.float32)
        return carry
    lax.fori_loop(0, PC * VECS_PER_ROW, zfill, None)

    def zfire(j, carry):
        idx = wid * PER_W + j
        h = 1 + idx // NZCHUNKS
        pc = idx % NZCHUNKS
        pltpu.async_copy(zeros_v, out_hbm.at[h, pl.ds(pc * PC, PC)], zsem)
        return carry
    lax.fori_loop(0, PER_W, zfire, None)

    base = wid * MEANP
    pltpu.async_copy(state_hbm.at[0, pl.ds(base, MEANP)], acc_v, ssem).wait()
    for b in range(1, B):
        pltpu.async_copy(state_hbm.at[b, pl.ds(base, MEANP)], chunk_v, ssem).wait()

        def accum(t, carry):
            r = t // VECS_PER_ROW
            i = (t % VECS_PER_ROW) * LANES
            acc_v[r, pl.ds(i, LANES)] = (
                acc_v[r, pl.ds(i, LANES)] + chunk_v[r, pl.ds(i, LANES)])
            return carry
        lax.fori_loop(0, MEANP * VECS_PER_ROW, accum, None)

    scale = jnp.float32(1.0 / B)

    def rescale(t, carry):
        r = t // VECS_PER_ROW
        i = (t % VECS_PER_ROW) * LANES
        acc_v[r, pl.ds(i, LANES)] = acc_v[r, pl.ds(i, LANES)] * scale
        return carry
    lax.fori_loop(0, MEANP * VECS_PER_ROW, rescale, None)

    pltpu.async_copy(acc_v, out_hbm.at[0, pl.ds(base, MEANP)], ssem).wait()

    def zdrain(j, carry):
        pltpu.make_async_copy(
            zeros_v, out_hbm.at[1, pl.ds(0, PC)], zsem).wait()
        return carry
    lax.fori_loop(0, PER_W, zdrain, None)


def kernel(state):
    if state.ndim == 2:
        state = state[None, :, :]
    mesh = plsc.VectorSubcoreMesh(core_axis_name="c", subcore_axis_name="s")
    k = functools.partial(
        pl.kernel,
        mesh=mesh,
        out_type=jax.ShapeDtypeStruct((MAX_H, P, D), jnp.float32),
        scratch_types=[
            pltpu.VMEM((PC, D), jnp.float32),
            pltpu.VMEM((MEANP, D), jnp.float32),
            pltpu.VMEM((MEANP, D), jnp.float32),
            pltpu.SemaphoreType.DMA,
            pltpu.SemaphoreType.DMA,
        ],
    )(_sc_kernel)
    buf = k(state)
    return buf, jnp.asarray(1, dtype=jnp.int32)
